# Initial kernel scaffold; baseline (speedup 1.0000x reference)
import jax
import jax.numpy as jnp
from jax.experimental import pallas as pl


def kernel(token_ids, is_number, number_vals, segment_ids, pos_ids, token_table, num_w, num_b, seg_table, pos_table, gamma, beta):
    raise NotImplementedError("write your pallas kernel here")


# speedup vs baseline: 3.1780x; 3.1780x over previous
import jax
import jax.numpy as jnp
from jax.experimental import pallas as pl
from jax.experimental.pallas import tpu as pltpu


def _tc_body(tok_ref, isnum_ref, vals_ref, seg_ref, pos_ref,
             tokt_ref, aux_ref, segt_ref, post_ref, out_ref):
    TB = out_ref.shape[0]
    ids = tok_ref[0]
    sids = seg_ref[0]
    pids = pos_ref[0]
    isn = isnum_ref[0]
    vals = vals_ref[0]

    def emb_lookup(idx, table):
        n = table.shape[0]
        oh = (idx == jax.lax.broadcasted_iota(jnp.int32, (TB, n), 1)
              ).astype(jnp.float32)
        return jnp.dot(oh, table[...], preferred_element_type=jnp.float32)

    tok_emb = emb_lookup(ids, tokt_ref)
    seg_emb = emb_lookup(sids, segt_ref)
    pos_emb = emb_lookup(pids, post_ref)
    num_w = aux_ref[0:1, :]
    num_b = aux_ref[1:2, :]
    gamma = aux_ref[2:3, :]
    beta = aux_ref[3:4, :]
    num_emb = vals * num_w + num_b
    tok = jnp.where(isn != 0, num_emb, tok_emb)
    emb = tok + seg_emb + pos_emb
    mu = jnp.mean(emb, axis=1, keepdims=True)
    var = jnp.mean((emb - mu) ** 2, axis=1, keepdims=True)
    out_ref[...] = (emb - mu) * jax.lax.rsqrt(var + 1e-5) * gamma + beta


def kernel(token_ids, is_number, number_vals, segment_ids, pos_ids,
           token_table, num_w, num_b, seg_table, pos_table, gamma, beta):
    B, L = token_ids.shape
    V, D = token_table.shape
    S = seg_table.shape[0]
    N = B * L
    TB = 512
    NB = N // TB

    def prep(x, dtype):
        return x.astype(dtype).reshape(NB, TB, 1)

    tok = prep(token_ids, jnp.int32)
    isn = prep(is_number, jnp.int32)
    vals = prep(number_vals, jnp.float32)
    seg = prep(segment_ids, jnp.int32)
    pos = prep(pos_ids, jnp.int32)

    tokt = jnp.pad(token_table, ((0, 8 - V), (0, 0)))
    segt = jnp.pad(seg_table, ((0, -S % 8), (0, 0)))
    post = jnp.pad(pos_table, ((0, -S % 8), (0, 0)))
    aux = jnp.concatenate([
        jnp.stack([num_w, num_b, gamma, beta]),
        jnp.zeros((4, D), jnp.float32)], axis=0)

    idx_spec = pl.BlockSpec((1, TB, 1), lambda i: (i, 0, 0))
    full = lambda r: pl.BlockSpec((r, D), lambda i: (0, 0))
    out = pl.pallas_call(
        _tc_body,
        grid=(NB,),
        in_specs=[idx_spec] * 5
                 + [full(8), full(8), full(segt.shape[0]), full(post.shape[0])],
        out_specs=pl.BlockSpec((TB, D), lambda i: (i, 0)),
        out_shape=jax.ShapeDtypeStruct((N, D), jnp.float32),
        compiler_params=pltpu.CompilerParams(
            dimension_semantics=("arbitrary",)),
    )(tok, isn, vals, seg, pos, tokt, aux, segt, post)
    return out.reshape(B, L, D)
